# baseline (speedup 1.0000x reference)
import functools

import jax
import jax.numpy as jnp
from jax import lax
from jax.experimental import pallas as pl
from jax.experimental.pallas import tpu as pltpu
from jax.experimental.pallas import tpu_sc as plsc

_BATCH = 16384
_D = 16
_NC = 2
_NS = 16
_NW = _NC * _NS
_BPW = _BATCH // _NW
_CHUNK = 128
_NCHUNK = _BPW // _CHUNK


def _efm_body(user_hbm, item_hbm, ue_hbm, ie_hbm, uhe_hbm, ihe_hbm, out_hbm,
              idx_u, idx_i, u_rows, i_rows, uh_rows, ih_rows, out_v, sem):
    wid = lax.axis_index("s") * _NC + lax.axis_index("c")
    base = wid * _BPW

    pltpu.sync_copy(user_hbm.at[pl.ds(base, _BPW)], idx_u)
    pltpu.sync_copy(item_hbm.at[pl.ds(base, _BPW)], idx_i)

    copies = []
    for j in range(_NCHUNK):
        sl = pl.ds(j * _CHUNK, _CHUNK)
        copies.append(pltpu.async_copy(ue_hbm.at[idx_u.at[sl]], u_rows.at[sl], sem))
        copies.append(pltpu.async_copy(ie_hbm.at[idx_i.at[sl]], i_rows.at[sl], sem))
        copies.append(pltpu.async_copy(uhe_hbm.at[idx_u.at[sl]], uh_rows.at[sl], sem))
        copies.append(pltpu.async_copy(ihe_hbm.at[idx_i.at[sl]], ih_rows.at[sl], sem))
    for c in copies:
        c.wait()

    lane = lax.iota(jnp.int32, 16)

    def group_body(g, _):
        base_e = g * 16
        acc = jnp.zeros((16,), jnp.float32)
        for r in range(16):
            e = base_e + r
            p = (u_rows[e, :] * i_rows[e, :]
                 + uh_rows[e, :] * ih_rows[e, :])
            s = jnp.sum(p)
            acc = jnp.where(lane == r, s, acc)
        out_v[pl.ds(base_e, 16)] = acc
        return 0

    lax.fori_loop(0, _BPW // 16, group_body, 0)

    pltpu.sync_copy(out_v, out_hbm.at[pl.ds(base, _BPW)])


@jax.jit
def kernel(user, item, user_emb, item_emb, user_h_emb, item_h_emb):
    mesh = plsc.VectorSubcoreMesh(core_axis_name="c", subcore_axis_name="s")
    run = pl.kernel(
        _efm_body,
        out_type=jax.ShapeDtypeStruct((_BATCH,), jnp.float32),
        mesh=mesh,
        scratch_types=[
            pltpu.VMEM((_BPW,), jnp.int32),
            pltpu.VMEM((_BPW,), jnp.int32),
            pltpu.VMEM((_BPW, _D), jnp.float32),
            pltpu.VMEM((_BPW, _D), jnp.float32),
            pltpu.VMEM((_BPW, _D), jnp.float32),
            pltpu.VMEM((_BPW, _D), jnp.float32),
            pltpu.VMEM((_BPW,), jnp.float32),
            pltpu.SemaphoreType.DMA,
        ],
        compiler_params=pltpu.CompilerParams(
            needs_layout_passes=False, use_tc_tiling_on_sc=False),
    )
    return run(user, item, user_emb, item_emb, user_h_emb, item_h_emb)
